# baseline (speedup 1.0000x reference)
import jax
import jax.numpy as jnp
from jax.experimental import pallas as pl
from jax.experimental.pallas import tpu as pltpu

_POOL = 7
_STRIDE = 8
_HW = 256
_NB_ROWS = _HW * _HW + 512
_B1 = 16
_B2 = 800


def _pool_kernel(
    p0_ref, p1_ref, wy_ref, wx_ref, slab_hbm, out_ref, s_ref, t_ref, sem
):
    h = pl.program_id(0)

    @pl.when(pl.program_id(1) == 0)
    def _load():
        dma = pltpu.make_async_copy(slab_hbm.at[h], s_ref, sem)
        dma.start()
        dma.wait()

    def body(nl, _):
        for b in range(_POOL * _POOL):
            p0 = p0_ref[0, nl, b]
            p1 = p1_ref[0, nl, b]
            wy = wy_ref[0, nl, b]
            wx = wx_ref[0, nl, b]
            a0 = pl.multiple_of((p0 >> 3) << 3, 8)
            a1 = pl.multiple_of((p1 >> 3) << 3, 8)
            c0 = s_ref[pl.ds(a0, 16), :]
            c1 = s_ref[pl.ds(a1, 16), :]
            r0 = pltpu.roll(c0, -(p0 & 7), axis=0)
            r1 = pltpu.roll(c1, -(p1 & 7), axis=0)
            vx = r0[0:2, :] * (1.0 - wx) + r1[0:2, :] * wx
            t_ref[b : b + 1, :] = vx[0:1, :] * (1.0 - wy) + vx[1:2, :] * wy
        t = t_ref[0 : _POOL * _POOL, :]
        lane = jax.lax.broadcasted_iota(jnp.int32, (_POOL * _POOL, 128), 1)
        row = jax.lax.broadcasted_iota(jnp.int32, (_POOL * _POOL, 128), 0)
        msk = ((lane % 49 == row) & (lane < 98)).astype(jnp.float32)
        out_ref[pl.ds(nl, 1), 0, :] = jnp.sum(t * msk, axis=0, keepdims=True)
        return 0

    jax.lax.fori_loop(0, _B1, body, 0)


def _head_kernel(p_ref, c0_ref, c1_ref, res_ref, sc_ref, m0_ref, m1_ref):
    x = p_ref[...]
    o_ofs = (0, 49, 128, 177)
    a = [x[:, o : o + 49] for o in (0, 49)] + [
        x[:, o : o + 49] for o in (128, 177)
    ]
    ave = []
    masks = []
    for m in range(2):
        l0, l1 = a[m], a[2 + m]
        mx = jnp.maximum(l0, l1)
        e0 = jnp.exp(l0 - mx)
        e1 = jnp.exp(l1 - mx)
        den = e0 + e1
        masks.append((e0 / den, e1 / den))
        ave.append(jnp.mean(mx, axis=1, keepdims=True))
    mxa = jnp.maximum(ave[0], ave[1])
    ea0 = jnp.exp(ave[0] - mxa)
    ea1 = jnp.exp(ave[1] - mxa)
    dena = ea0 + ea1
    s0 = ea0 / dena
    s1 = ea1 / dena
    res = (s1 > s0)
    c0_ref[...] = s0
    c1_ref[...] = s1
    res_ref[...] = res.astype(jnp.int32)
    sc_ref[...] = jnp.maximum(s0, s1)
    m0_ref[...] = jnp.where(res, masks[1][0], masks[0][0])
    m1_ref[...] = jnp.where(res, masks[1][1], masks[0][1])


def kernel(roi, ps_score_map):
    k = _POOL
    n = roi.shape[0]
    sm = ps_score_map[0]
    s = jnp.transpose(sm, (1, 0, 2)).reshape(_HW * _HW, 4 * k * k)
    s = jnp.pad(s, ((0, _NB_ROWS - _HW * _HW), (0, 0)))
    slab = jnp.stack(
        [
            jnp.pad(s[:, 0:98], ((0, 0), (0, 30))),
            jnp.pad(s[:, 98:196], ((0, 0), (0, 30))),
        ]
    )

    f = roi.astype(jnp.float32) / _STRIDE
    x1, y1, x2, y2 = f[:, 0], f[:, 1], f[:, 2], f[:, 3]
    roi_w = jnp.maximum(x2 - x1, 1.0)
    roi_h = jnp.maximum(y2 - y1, 1.0)
    cc = (jnp.arange(k, dtype=jnp.float32) + 0.5) / k
    cy = jnp.clip(y1[:, None] + cc[None, :] * roi_h[:, None], 0.0, _HW - 1.0)
    cx = jnp.clip(x1[:, None] + cc[None, :] * roi_w[:, None], 0.0, _HW - 1.0)
    y0f = jnp.floor(cy)
    x0f = jnp.floor(cx)
    wy = (cy - y0f)[:, :, None, None]
    wx = (cx - x0f)[:, None, :, None]
    y0 = y0f.astype(jnp.int32)
    x0 = x0f.astype(jnp.int32)
    x1i = jnp.minimum(x0 + 1, _HW - 1)
    p0 = (x0[:, None, :] * _HW + y0[:, :, None]).reshape(n, k * k)
    p1 = (x1i[:, None, :] * _HW + y0[:, :, None]).reshape(n, k * k)
    wyb = jnp.broadcast_to(wy, (n, k, k, 1)).reshape(n, k * k)
    wxb = jnp.broadcast_to(wx, (n, k, k, 1)).reshape(n, k * k)

    nb = n // _B1
    p0 = p0.reshape(nb, _B1, k * k)
    p1 = p1.reshape(nb, _B1, k * k)
    wyb = wyb.reshape(nb, _B1, k * k)
    wxb = wxb.reshape(nb, _B1, k * k)

    smem = lambda: pl.BlockSpec(
        (1, _B1, k * k), lambda h, i: (i, 0, 0), memory_space=pltpu.SMEM
    )
    pooled = pl.pallas_call(
        _pool_kernel,
        grid=(2, nb),
        in_specs=[
            smem(),
            smem(),
            smem(),
            smem(),
            pl.BlockSpec(memory_space=pl.ANY),
        ],
        out_specs=pl.BlockSpec(
            (_B1, 1, 128), lambda h, i: (h * nb + i, 0, 0)
        ),
        out_shape=jax.ShapeDtypeStruct((2 * n, 1, 128), jnp.float32),
        scratch_shapes=[
            pltpu.VMEM((_NB_ROWS, 128), jnp.float32),
            pltpu.VMEM((56, 128), jnp.float32),
            pltpu.SemaphoreType.DMA,
        ],
        compiler_params=pltpu.CompilerParams(
            dimension_semantics=("parallel", "arbitrary"),
            vmem_limit_bytes=56 * 1024 * 1024,
        ),
    )(p0, p1, wyb, wxb, slab)

    ph = pooled.reshape(2, n, 128)
    prow = jnp.concatenate([ph[0], ph[1]], axis=-1)

    nb2 = n // _B2
    blk = lambda w, d: pl.BlockSpec((_B2, w), lambda i: (i, 0))
    outs = pl.pallas_call(
        _head_kernel,
        grid=(nb2,),
        in_specs=[pl.BlockSpec((_B2, 256), lambda i: (i, 0))],
        out_specs=[
            blk(1, jnp.float32),
            blk(1, jnp.float32),
            blk(1, jnp.int32),
            blk(1, jnp.float32),
            blk(49, jnp.float32),
            blk(49, jnp.float32),
        ],
        out_shape=[
            jax.ShapeDtypeStruct((n, 1), jnp.float32),
            jax.ShapeDtypeStruct((n, 1), jnp.float32),
            jax.ShapeDtypeStruct((n, 1), jnp.int32),
            jax.ShapeDtypeStruct((n, 1), jnp.float32),
            jax.ShapeDtypeStruct((n, 49), jnp.float32),
            jax.ShapeDtypeStruct((n, 49), jnp.float32),
        ],
        compiler_params=pltpu.CompilerParams(
            dimension_semantics=("parallel",)
        ),
    )(prow)
    c0, c1, res, sc, m0, m1 = outs
    cls = jnp.concatenate([c0, c1], axis=1)
    mask_result = jnp.stack([m0, m1], axis=-1).reshape(n, k, k, 2)
    return cls, res, sc[:, 0], mask_result


# speedup vs baseline: 1.8415x; 1.8415x over previous
import jax
import jax.numpy as jnp
from jax.experimental import pallas as pl
from jax.experimental.pallas import tpu as pltpu

_POOL = 7
_STRIDE = 8
_HW = 256
_NB_ROWS = _HW * _HW + 512
_BH = 25
_B1 = 8
_B2 = 800


def _pool_kernel(p0_ref, p1_ref, wy_ref, wx_ref, slab_hbm, out_ref,
                 s_ref, t_ref, sem):
    h = pl.program_id(0)

    @pl.when(pl.program_id(1) == 0)
    def _load():
        dma = pltpu.make_async_copy(slab_hbm.at[h], s_ref, sem)
        dma.start()
        dma.wait()

    lane = jax.lax.broadcasted_iota(jnp.int32, (_BH, 128), 1)
    row = jax.lax.broadcasted_iota(jnp.int32, (_BH, 128), 0)
    msk = ((lane % _BH == row) & (lane < 4 * _BH)).astype(jnp.float32)
    for nl in range(_B1):
        for b in range(_BH):
            p0 = p0_ref[0, nl, b]
            p1 = p1_ref[0, nl, b]
            wy = wy_ref[0, nl, b]
            wx = wx_ref[0, nl, b]
            a0 = pl.multiple_of((p0 >> 3) << 3, 8)
            a1 = pl.multiple_of((p1 >> 3) << 3, 8)
            c0 = s_ref[pl.ds(a0, 16), :]
            c1 = s_ref[pl.ds(a1, 16), :]
            r0 = pltpu.roll(c0, -(p0 & 7), axis=0)
            r1 = pltpu.roll(c1, -(p1 & 7), axis=0)
            vx = r0[0:2, :] * (1.0 - wx) + r1[0:2, :] * wx
            t_ref[b : b + 1, :] = vx[0:1, :] * (1.0 - wy) + vx[1:2, :] * wy
        t = t_ref[0:_BH, :]
        out_ref[nl : nl + 1, 0, :] = jnp.sum(t * msk, axis=0, keepdims=True)


def _head_kernel(p_ref, c0_ref, c1_ref, res_ref, sc_ref, m0_ref, m1_ref):
    x = p_ref[...]
    a = [x[:, o : o + 49] for o in (0, 49, 128, 177)]
    ave = []
    masks = []
    for m in range(2):
        l0, l1 = a[m], a[2 + m]
        mx = jnp.maximum(l0, l1)
        e0 = jnp.exp(l0 - mx)
        e1 = jnp.exp(l1 - mx)
        den = e0 + e1
        masks.append((e0 / den, e1 / den))
        ave.append(jnp.mean(mx, axis=1, keepdims=True))
    mxa = jnp.maximum(ave[0], ave[1])
    ea0 = jnp.exp(ave[0] - mxa)
    ea1 = jnp.exp(ave[1] - mxa)
    dena = ea0 + ea1
    s0 = ea0 / dena
    s1 = ea1 / dena
    res = (s1 > s0)
    c0_ref[...] = s0
    c1_ref[...] = s1
    res_ref[...] = res.astype(jnp.int32)
    sc_ref[...] = jnp.maximum(s0, s1)
    m0_ref[...] = jnp.where(res, masks[1][0], masks[0][0])
    m1_ref[...] = jnp.where(res, masks[1][1], masks[0][1])


def kernel(roi, ps_score_map):
    k = _POOL
    n = roi.shape[0]
    sm = ps_score_map[0]
    s = jnp.transpose(sm, (1, 0, 2)).reshape(_HW * _HW, 4 * k * k)
    s = jnp.pad(s, ((0, _NB_ROWS - _HW * _HW), (0, 0)))
    cols = []
    for b0 in (0, _BH - 1):
        cols.append(
            [o * 49 + b0 + lb for o in range(4) for lb in range(_BH)]
        )
    slab = jnp.stack(
        [jnp.pad(s[:, jnp.array(c)], ((0, 0), (0, 28))) for c in cols]
    )

    f = roi.astype(jnp.float32) / _STRIDE
    x1, y1, x2, y2 = f[:, 0], f[:, 1], f[:, 2], f[:, 3]
    roi_w = jnp.maximum(x2 - x1, 1.0)
    roi_h = jnp.maximum(y2 - y1, 1.0)
    cc = (jnp.arange(k, dtype=jnp.float32) + 0.5) / k
    cy = jnp.clip(y1[:, None] + cc[None, :] * roi_h[:, None], 0.0, _HW - 1.0)
    cx = jnp.clip(x1[:, None] + cc[None, :] * roi_w[:, None], 0.0, _HW - 1.0)
    y0f = jnp.floor(cy)
    x0f = jnp.floor(cx)
    wy = (cy - y0f)[:, :, None]
    wx = (cx - x0f)[:, None, :]
    y0 = y0f.astype(jnp.int32)
    x0 = x0f.astype(jnp.int32)
    x1i = jnp.minimum(x0 + 1, _HW - 1)
    p0 = (x0[:, None, :] * _HW + y0[:, :, None]).reshape(n, k * k)
    p1 = (x1i[:, None, :] * _HW + y0[:, :, None]).reshape(n, k * k)
    wyb = jnp.broadcast_to(wy, (n, k, k)).reshape(n, k * k)
    wxb = jnp.broadcast_to(wx, (n, k, k)).reshape(n, k * k)

    nb = n // _B1

    def split(arr):
        both = jnp.stack([arr[:, :_BH], arr[:, _BH - 1 : 2 * _BH - 1]])
        return both.reshape(2 * nb, _B1, _BH)

    p0s, p1s, wys, wxs = split(p0), split(p1), split(wyb), split(wxb)

    smem = lambda: pl.BlockSpec(
        (1, _B1, _BH), lambda h, i: (h * nb + i, 0, 0),
        memory_space=pltpu.SMEM,
    )
    pooled = pl.pallas_call(
        _pool_kernel,
        grid=(2, nb),
        in_specs=[
            smem(),
            smem(),
            smem(),
            smem(),
            pl.BlockSpec(memory_space=pl.ANY),
        ],
        out_specs=pl.BlockSpec(
            (_B1, 1, 128), lambda h, i: (h * nb + i, 0, 0)
        ),
        out_shape=jax.ShapeDtypeStruct((2 * n, 1, 128), jnp.float32),
        scratch_shapes=[
            pltpu.VMEM((_NB_ROWS, 128), jnp.float32),
            pltpu.VMEM((32, 128), jnp.float32),
            pltpu.SemaphoreType.DMA,
        ],
        compiler_params=pltpu.CompilerParams(
            dimension_semantics=("parallel", "arbitrary"),
            vmem_limit_bytes=56 * 1024 * 1024,
        ),
    )(p0s, p1s, wys, wxs, slab)

    ph = pooled.reshape(2, n, 128)
    po = []
    for o in range(4):
        po.append(ph[0][:, o * _BH : o * _BH + _BH])
        po.append(ph[1][:, o * _BH + 1 : o * _BH + _BH])
    z = jnp.zeros((n, 30), jnp.float32)
    prow = jnp.concatenate(po[0:4] + [z] + po[4:8] + [z], axis=-1)

    nb2 = n // _B2
    blk = lambda w: pl.BlockSpec((_B2, w), lambda i: (i, 0))
    outs = pl.pallas_call(
        _head_kernel,
        grid=(nb2,),
        in_specs=[pl.BlockSpec((_B2, 256), lambda i: (i, 0))],
        out_specs=[blk(1), blk(1), blk(1), blk(1), blk(49), blk(49)],
        out_shape=[
            jax.ShapeDtypeStruct((n, 1), jnp.float32),
            jax.ShapeDtypeStruct((n, 1), jnp.float32),
            jax.ShapeDtypeStruct((n, 1), jnp.int32),
            jax.ShapeDtypeStruct((n, 1), jnp.float32),
            jax.ShapeDtypeStruct((n, 49), jnp.float32),
            jax.ShapeDtypeStruct((n, 49), jnp.float32),
        ],
        compiler_params=pltpu.CompilerParams(
            dimension_semantics=("parallel",)
        ),
    )(prow)
    c0, c1, res, sc, m0, m1 = outs
    cls = jnp.concatenate([c0, c1], axis=1)
    mask_result = jnp.stack([m0, m1], axis=-1).reshape(n, k, k, 2)
    return cls, res, sc[:, 0], mask_result


# speedup vs baseline: 1.8772x; 1.0194x over previous
import jax
import jax.numpy as jnp
from jax.experimental import pallas as pl
from jax.experimental.pallas import tpu as pltpu

_POOL = 7
_STRIDE = 8
_HW = 256
_NB_ROWS = _HW * _HW + 512
_BH = 25
_B1 = 16
_B2 = 800


def _pool_kernel(p0_ref, p1_ref, wy_ref, wx_ref, slab_hbm, out_ref,
                 s_ref, t_ref, sem):
    h = pl.program_id(0)

    @pl.when(pl.program_id(1) == 0)
    def _load():
        dma = pltpu.make_async_copy(slab_hbm.at[h], s_ref, sem)
        dma.start()
        dma.wait()

    lane = jax.lax.broadcasted_iota(jnp.int32, (_BH, 128), 1)
    row = jax.lax.broadcasted_iota(jnp.int32, (_BH, 128), 0)
    msk = ((lane % _BH == row) & (lane < 4 * _BH)).astype(jnp.float32)
    for nl in range(_B1):
        for b in range(_BH):
            p0 = p0_ref[0, nl, b]
            p1 = p1_ref[0, nl, b]
            wy = wy_ref[0, nl, b]
            wx = wx_ref[0, nl, b]
            a0 = pl.multiple_of((p0 >> 3) << 3, 8)
            a1 = pl.multiple_of((p1 >> 3) << 3, 8)
            c0 = s_ref[pl.ds(a0, 16), :]
            c1 = s_ref[pl.ds(a1, 16), :]
            r0 = pltpu.roll(c0, -(p0 & 7), axis=0)
            r1 = pltpu.roll(c1, -(p1 & 7), axis=0)
            vx = r0[0:2, :] * (1.0 - wx) + r1[0:2, :] * wx
            t_ref[b : b + 1, :] = vx[0:1, :] * (1.0 - wy) + vx[1:2, :] * wy
        t = t_ref[0:_BH, :]
        out_ref[nl : nl + 1, 0, :] = jnp.sum(t * msk, axis=0, keepdims=True)


def _head_kernel(p_ref, c0_ref, c1_ref, res_ref, sc_ref, m0_ref, m1_ref):
    x = p_ref[...]
    a = [x[:, o : o + 49] for o in (0, 49, 128, 177)]
    ave = []
    masks = []
    for m in range(2):
        l0, l1 = a[m], a[2 + m]
        mx = jnp.maximum(l0, l1)
        e0 = jnp.exp(l0 - mx)
        e1 = jnp.exp(l1 - mx)
        den = e0 + e1
        masks.append((e0 / den, e1 / den))
        ave.append(jnp.mean(mx, axis=1, keepdims=True))
    mxa = jnp.maximum(ave[0], ave[1])
    ea0 = jnp.exp(ave[0] - mxa)
    ea1 = jnp.exp(ave[1] - mxa)
    dena = ea0 + ea1
    s0 = ea0 / dena
    s1 = ea1 / dena
    res = (s1 > s0)
    c0_ref[...] = s0
    c1_ref[...] = s1
    res_ref[...] = res.astype(jnp.int32)
    sc_ref[...] = jnp.maximum(s0, s1)
    m0_ref[...] = jnp.where(res, masks[1][0], masks[0][0])
    m1_ref[...] = jnp.where(res, masks[1][1], masks[0][1])


def kernel(roi, ps_score_map):
    k = _POOL
    n = roi.shape[0]
    sm = ps_score_map[0]
    s = jnp.transpose(sm, (1, 0, 2)).reshape(_HW * _HW, 4 * k * k)
    s = jnp.pad(s, ((0, _NB_ROWS - _HW * _HW), (0, 0)))
    cols = []
    for b0 in (0, _BH - 1):
        cols.append(
            [o * 49 + b0 + lb for o in range(4) for lb in range(_BH)]
        )
    slab = jnp.stack(
        [jnp.pad(s[:, jnp.array(c)], ((0, 0), (0, 28))) for c in cols]
    )

    f = roi.astype(jnp.float32) / _STRIDE
    x1, y1, x2, y2 = f[:, 0], f[:, 1], f[:, 2], f[:, 3]
    roi_w = jnp.maximum(x2 - x1, 1.0)
    roi_h = jnp.maximum(y2 - y1, 1.0)
    cc = (jnp.arange(k, dtype=jnp.float32) + 0.5) / k
    cy = jnp.clip(y1[:, None] + cc[None, :] * roi_h[:, None], 0.0, _HW - 1.0)
    cx = jnp.clip(x1[:, None] + cc[None, :] * roi_w[:, None], 0.0, _HW - 1.0)
    y0f = jnp.floor(cy)
    x0f = jnp.floor(cx)
    wy = (cy - y0f)[:, :, None]
    wx = (cx - x0f)[:, None, :]
    y0 = y0f.astype(jnp.int32)
    x0 = x0f.astype(jnp.int32)
    x1i = jnp.minimum(x0 + 1, _HW - 1)
    p0 = (x0[:, None, :] * _HW + y0[:, :, None]).reshape(n, k * k)
    p1 = (x1i[:, None, :] * _HW + y0[:, :, None]).reshape(n, k * k)
    wyb = jnp.broadcast_to(wy, (n, k, k)).reshape(n, k * k)
    wxb = jnp.broadcast_to(wx, (n, k, k)).reshape(n, k * k)

    nb = n // _B1

    def split(arr):
        both = jnp.stack([arr[:, :_BH], arr[:, _BH - 1 : 2 * _BH - 1]])
        return both.reshape(2 * nb, _B1, _BH)

    p0s, p1s, wys, wxs = split(p0), split(p1), split(wyb), split(wxb)

    smem = lambda: pl.BlockSpec(
        (1, _B1, _BH), lambda h, i: (h * nb + i, 0, 0),
        memory_space=pltpu.SMEM,
    )
    pooled = pl.pallas_call(
        _pool_kernel,
        grid=(2, nb),
        in_specs=[
            smem(),
            smem(),
            smem(),
            smem(),
            pl.BlockSpec(memory_space=pl.ANY),
        ],
        out_specs=pl.BlockSpec(
            (_B1, 1, 128), lambda h, i: (h * nb + i, 0, 0)
        ),
        out_shape=jax.ShapeDtypeStruct((2 * n, 1, 128), jnp.float32),
        scratch_shapes=[
            pltpu.VMEM((_NB_ROWS, 128), jnp.float32),
            pltpu.VMEM((32, 128), jnp.float32),
            pltpu.SemaphoreType.DMA,
        ],
        compiler_params=pltpu.CompilerParams(
            dimension_semantics=("parallel", "arbitrary"),
            vmem_limit_bytes=56 * 1024 * 1024,
        ),
    )(p0s, p1s, wys, wxs, slab)

    ph = pooled.reshape(2, n, 128)
    po = []
    for o in range(4):
        po.append(ph[0][:, o * _BH : o * _BH + _BH])
        po.append(ph[1][:, o * _BH + 1 : o * _BH + _BH])
    z = jnp.zeros((n, 30), jnp.float32)
    prow = jnp.concatenate(po[0:4] + [z] + po[4:8] + [z], axis=-1)

    nb2 = n // _B2
    blk = lambda w: pl.BlockSpec((_B2, w), lambda i: (i, 0))
    outs = pl.pallas_call(
        _head_kernel,
        grid=(nb2,),
        in_specs=[pl.BlockSpec((_B2, 256), lambda i: (i, 0))],
        out_specs=[blk(1), blk(1), blk(1), blk(1), blk(49), blk(49)],
        out_shape=[
            jax.ShapeDtypeStruct((n, 1), jnp.float32),
            jax.ShapeDtypeStruct((n, 1), jnp.float32),
            jax.ShapeDtypeStruct((n, 1), jnp.int32),
            jax.ShapeDtypeStruct((n, 1), jnp.float32),
            jax.ShapeDtypeStruct((n, 49), jnp.float32),
            jax.ShapeDtypeStruct((n, 49), jnp.float32),
        ],
        compiler_params=pltpu.CompilerParams(
            dimension_semantics=("parallel",)
        ),
    )(prow)
    c0, c1, res, sc, m0, m1 = outs
    cls = jnp.concatenate([c0, c1], axis=1)
    mask_result = jnp.stack([m0, m1], axis=-1).reshape(n, k, k, 2)
    return cls, res, sc[:, 0], mask_result
